# baseline (device time: 72320 ns/iter reference)
import jax
import jax.numpy as jnp
from jax import lax
from jax.experimental import pallas as pl
from jax.experimental.pallas import tpu as pltpu


def kernel(x):
    m, n = x.shape
    half = n // 2

    import os
    vmem_fill = int(os.environ.get("VMEM_FILL_BYTES", "0"))

    x = pltpu.with_memory_space_constraint(x, pltpu.MemorySpace.HBM)

    def body(x_ref, out_ref, send_sem, recv_sem, local_sem, *maybe_fill):
        my_x = lax.axis_index("x")
        my_y = lax.axis_index("y")
        my_z = lax.axis_index("z")
        partner = (my_x, my_y, 1 - my_z)

        barrier_sem = pltpu.get_barrier_semaphore()
        pl.semaphore_signal(
            barrier_sem, inc=1,
            device_id=partner, device_id_type=pl.DeviceIdType.MESH,
        )
        pl.semaphore_wait(barrier_sem, 1)

        rdma = pltpu.make_async_remote_copy(
            src_ref=x_ref.at[:, pl.ds((1 - my_z) * half, half)],
            dst_ref=out_ref.at[pl.ds(my_z * m, m), :],
            send_sem=send_sem,
            recv_sem=recv_sem,
            device_id=partner,
            device_id_type=pl.DeviceIdType.MESH,
        )
        rdma.start()
        local = pltpu.make_async_copy(
            x_ref.at[:, pl.ds(my_z * half, half)],
            out_ref.at[pl.ds(my_z * m, m), :],
            local_sem,
        )
        local.start()
        local.wait()
        rdma.wait()

    return pl.pallas_call(
        body,
        out_shape=pltpu.MemorySpace.HBM((2 * m, half), jnp.float32),
        in_specs=[pl.BlockSpec(memory_space=pltpu.MemorySpace.HBM)],
        out_specs=pl.BlockSpec(memory_space=pltpu.MemorySpace.HBM),
        scratch_shapes=[
            pltpu.SemaphoreType.DMA,
            pltpu.SemaphoreType.DMA,
            pltpu.SemaphoreType.DMA,
        ] + ([pltpu.VMEM((vmem_fill // 4,), jnp.float32)] if vmem_fill else []),
        compiler_params=pltpu.CompilerParams(collective_id=0),
    )(x)


# device time: 29594 ns/iter; 2.4437x vs baseline; 2.4437x over previous
import jax
import jax.numpy as jnp
from jax import lax
from jax.experimental import pallas as pl
from jax.experimental.pallas import tpu as pltpu

N_CHUNKS = 4


def kernel(x):
    m, n = x.shape
    half = n // 2
    rows_per = m // N_CHUNKS

    x = pltpu.with_memory_space_constraint(x, pltpu.MemorySpace.HBM)

    def body(x_ref, out_ref, stage_ref, send_sems, recv_sems, stage_sems,
             local_sem):
        my_x = lax.axis_index("x")
        my_y = lax.axis_index("y")
        my_z = lax.axis_index("z")
        partner = (my_x, my_y, 1 - my_z)

        barrier_sem = pltpu.get_barrier_semaphore()
        pl.semaphore_signal(
            barrier_sem, inc=1,
            device_id=partner, device_id_type=pl.DeviceIdType.MESH,
        )

        local = pltpu.make_async_copy(
            x_ref.at[:, pl.ds(my_z * half, half)],
            out_ref.at[pl.ds(my_z * m, m), :],
            local_sem,
        )
        local.start()

        stages = []
        for c in range(N_CHUNKS):
            st = pltpu.make_async_copy(
                x_ref.at[pl.ds(c * rows_per, rows_per),
                         pl.ds((1 - my_z) * half, half)],
                stage_ref.at[pl.ds(c * rows_per, rows_per), :],
                stage_sems.at[c],
            )
            st.start()
            stages.append(st)

        pl.semaphore_wait(barrier_sem, 1)

        rdmas = []
        for c in range(N_CHUNKS):
            stages[c].wait()
            rdma = pltpu.make_async_remote_copy(
                src_ref=stage_ref.at[pl.ds(c * rows_per, rows_per), :],
                dst_ref=out_ref.at[pl.ds(my_z * m + c * rows_per, rows_per), :],
                send_sem=send_sems.at[c],
                recv_sem=recv_sems.at[c],
                device_id=partner,
                device_id_type=pl.DeviceIdType.MESH,
            )
            rdma.start()
            rdmas.append(rdma)

        local.wait()
        for rdma in rdmas:
            rdma.wait()

    return pl.pallas_call(
        body,
        out_shape=jax.ShapeDtypeStruct((2 * m, half), jnp.float32),
        in_specs=[pl.BlockSpec(memory_space=pltpu.MemorySpace.HBM)],
        out_specs=pl.BlockSpec(memory_space=pltpu.MemorySpace.VMEM),
        scratch_shapes=[
            pltpu.VMEM((m, half), jnp.float32),
            pltpu.SemaphoreType.DMA((N_CHUNKS,)),
            pltpu.SemaphoreType.DMA((N_CHUNKS,)),
            pltpu.SemaphoreType.DMA((N_CHUNKS,)),
            pltpu.SemaphoreType.DMA,
        ],
        compiler_params=pltpu.CompilerParams(collective_id=0),
    )(x)
